# Initial kernel scaffold; baseline (speedup 1.0000x reference)
import jax
import jax.numpy as jnp
from jax.experimental import pallas as pl


def kernel(nxyz, nbr_list, params):
    raise NotImplementedError("write your pallas kernel here")


# speedup vs baseline: 16.7321x; 16.7321x over previous
import functools

import jax
import jax.numpy as jnp
from jax import lax
from jax.experimental import pallas as pl
from jax.experimental.pallas import tpu as pltpu
from jax.experimental.pallas import tpu_sc as plsc

N = 10000
EU = 320000
E2 = 2 * EU
F = 128
RBF = 20
CUT = 5.0
OUT_FEA = 64

NB = 400
EB = 512
GB = 80
NW = 32
NSUB = 16


def _silu(x):
    return x / (1.0 + jnp.exp(-x))


def _embed_kernel(nxyz_ref, emb_ref, o_ref):
    z = nxyz_ref[:, 0].astype(jnp.int32)
    cols = lax.broadcasted_iota(jnp.int32, (NB, F), 1)
    onehot = (cols == z[:, None]).astype(jnp.float32)
    o_ref[...] = jnp.dot(onehot, emb_ref[...],
                         preferred_element_type=jnp.float32)


def _embed(nxyz, emb_pad):
    return pl.pallas_call(
        _embed_kernel,
        grid=(N // NB,),
        in_specs=[pl.BlockSpec((NB, 4), lambda i: (i, 0)),
                  pl.BlockSpec((F, F), lambda i: (0, 0))],
        out_specs=pl.BlockSpec((NB, F), lambda i: (i, 0)),
        out_shape=jax.ShapeDtypeStruct((N, F), jnp.float32),
    )(nxyz, emb_pad)


def _geom_kernel(xs_ref, xd_ref, p_ref, u_ref):
    r8 = xs_ref[...] - xd_ref[...]
    d2 = jnp.sum(r8 * r8, axis=1, keepdims=True)
    dist = jnp.sqrt(d2 + 1e-15)
    u_ref[...] = (r8 / dist)[:, 0:16]
    fc = 0.5 * (jnp.cos(jnp.pi * dist / CUT) + 1.0)
    fc = fc * (dist <= CUT).astype(jnp.float32)
    k = lax.broadcasted_iota(jnp.int32, (EB, 32), 1)
    nfreq = (k + 1).astype(jnp.float32)
    rbf = jnp.sin(nfreq * (jnp.pi / CUT) * dist) / dist * fc
    p_ref[...] = jnp.where(k < RBF, rbf, jnp.where(k == RBF, fc, 0.0))


def _geometry(xyz_s, xyz_d):
    return pl.pallas_call(
        _geom_kernel,
        grid=(EU // EB,),
        in_specs=[pl.BlockSpec((EB, 128), lambda i: (i, 0)),
                  pl.BlockSpec((EB, 128), lambda i: (i, 0))],
        out_specs=[pl.BlockSpec((EB, 32), lambda i: (i, 0)),
                   pl.BlockSpec((EB, 16), lambda i: (i, 0))],
        out_shape=[jax.ShapeDtypeStruct((EU, 32), jnp.float32),
                   jax.ShapeDtypeStruct((EU, 16), jnp.float32)],
    )(xyz_s, xyz_d)


def _phi_kernel(s_ref, w1_ref, b1_ref, w2_ref, b2_ref, o_ref):
    h = _silu(jnp.dot(s_ref[...], w1_ref[...],
                      preferred_element_type=jnp.float32) + b1_ref[...])
    o_ref[...] = jnp.dot(h, w2_ref[...],
                         preferred_element_type=jnp.float32) + b2_ref[...]


def _phi(s, w1, b1, w2, b2):
    return pl.pallas_call(
        _phi_kernel,
        grid=(N // NB,),
        in_specs=[pl.BlockSpec((NB, F), lambda i: (i, 0)),
                  pl.BlockSpec((F, F), lambda i: (0, 0)),
                  pl.BlockSpec((1, F), lambda i: (0, 0)),
                  pl.BlockSpec((F, 3 * F), lambda i: (0, 0)),
                  pl.BlockSpec((1, 3 * F), lambda i: (0, 0))],
        out_specs=pl.BlockSpec((NB, 3 * F), lambda i: (i, 0)),
        out_shape=jax.ShapeDtypeStruct((N, 3 * F), jnp.float32),
    )(s, w1, b1, w2, b2)


def _edge_kernel(has_v, p_ref, u_ref, wd_ref, phig_ref, *rest):
    if has_v:
        vg_ref, m_ref = rest
    else:
        (m_ref,) = rest
    nblk = EU // EB
    sign = jnp.where(pl.program_id(0) < nblk, 1.0, -1.0)
    w_s = jnp.dot(p_ref[...], wd_ref[...],
                  preferred_element_type=jnp.float32)
    phig = phig_ref[...]
    inv0 = phig[:, 0:F] * w_s[:, 0:F]
    inv1 = phig[:, F:2 * F] * w_s[:, F:2 * F]
    inv2 = phig[:, 2 * F:3 * F] * w_s[:, 2 * F:3 * F]
    m_ref[:, 0:F] = inv1
    for d in range(3):
        unit_d = sign * u_ref[:, d][:, None]
        dv = inv2 * unit_d
        if has_v:
            dv = dv + inv0 * vg_ref[:, d * F:(d + 1) * F]
        m_ref[:, (d + 1) * F:(d + 2) * F] = dv


def _edge_messages(p_u, unit_u, wd, phig, vg):
    nblk = EU // EB
    has_v = vg is not None
    in_specs = [pl.BlockSpec((EB, 32), lambda i: (i % nblk, 0)),
                pl.BlockSpec((EB, 16), lambda i: (i % nblk, 0)),
                pl.BlockSpec((32, 3 * F), lambda i: (0, 0)),
                pl.BlockSpec((EB, 3 * F), lambda i: (i, 0))]
    args = [p_u, unit_u, wd, phig]
    if has_v:
        in_specs.append(pl.BlockSpec((EB, 3 * F), lambda i: (i, 0)))
        args.append(vg)
    return pl.pallas_call(
        functools.partial(_edge_kernel, has_v),
        grid=(E2 // EB,),
        in_specs=in_specs,
        out_specs=pl.BlockSpec((EB, 4 * F), lambda i: (i, 0)),
        out_shape=jax.ShapeDtypeStruct((E2, 4 * F), jnp.float32),
    )(*args)


def _update_kernel(s_ref, v_ref, m_ref, u_w_ref, v_w_ref,
                   w1_ref, b1_ref, w2_ref, b2_ref, so_ref, vo_ref):
    m = m_ref[...]
    s1 = s_ref[...] + m[0]
    v1 = [v_ref[:, d * F:(d + 1) * F] + m[1 + d] for d in range(3)]
    u_w = u_w_ref[...]
    v_w = v_w_ref[...]
    u_v = [jnp.dot(v1[d], u_w, preferred_element_type=jnp.float32)
           for d in range(3)]
    v_v = [jnp.dot(v1[d], v_w, preferred_element_type=jnp.float32)
           for d in range(3)]
    vvn = jnp.sqrt(v_v[0] * v_v[0] + v_v[1] * v_v[1] + v_v[2] * v_v[2]
                   + 1e-15)
    stack = jnp.concatenate([s1, vvn], axis=1)
    a = _silu(jnp.dot(stack, w1_ref[...],
                      preferred_element_type=jnp.float32) + b1_ref[...])
    split = jnp.dot(a, w2_ref[...],
                    preferred_element_type=jnp.float32) + b2_ref[...]
    s0 = split[:, 0:F]
    dot_uv = u_v[0] * v_v[0] + u_v[1] * v_v[1] + u_v[2] * v_v[2]
    so_ref[...] = s1 + split[:, F:2 * F] * dot_uv + split[:, 2 * F:3 * F]
    for d in range(3):
        vo_ref[:, d * F:(d + 1) * F] = v1[d] + u_v[d] * s0


def _update(s, v, msum, u_w, v_w, w1, b1, w2, b2):
    return pl.pallas_call(
        _update_kernel,
        grid=(N // NB,),
        in_specs=[pl.BlockSpec((NB, F), lambda i: (i, 0)),
                  pl.BlockSpec((NB, 3 * F), lambda i: (i, 0)),
                  pl.BlockSpec((4, NB, F), lambda i: (0, i, 0)),
                  pl.BlockSpec((F, F), lambda i: (0, 0)),
                  pl.BlockSpec((F, F), lambda i: (0, 0)),
                  pl.BlockSpec((2 * F, F), lambda i: (0, 0)),
                  pl.BlockSpec((1, F), lambda i: (0, 0)),
                  pl.BlockSpec((F, 3 * F), lambda i: (0, 0)),
                  pl.BlockSpec((1, 3 * F), lambda i: (0, 0))],
        out_specs=[pl.BlockSpec((NB, F), lambda i: (i, 0)),
                   pl.BlockSpec((NB, 3 * F), lambda i: (i, 0))],
        out_shape=[jax.ShapeDtypeStruct((N, F), jnp.float32),
                   jax.ShapeDtypeStruct((N, 3 * F), jnp.float32)],
    )(s, v, msum, u_w, v_w, w1, b1, w2, b2)


def _readout_kernel(s_ref, wr1_ref, br1_ref, wr2_ref, br2_ref,
                    wf1_ref, bf1_ref, wf2_ref, bf2_ref,
                    wo_ref, bo_ref, o_ref):
    h = _silu(jnp.dot(s_ref[...], wr1_ref[...],
                      preferred_element_type=jnp.float32) + br1_ref[...])
    af = jnp.dot(h, wr2_ref[...],
                 preferred_element_type=jnp.float32) + br2_ref[...]
    h = _silu(jnp.dot(af, wf1_ref[...],
                      preferred_element_type=jnp.float32) + bf1_ref[...])
    h = _silu(jnp.dot(h, wf2_ref[...],
                      preferred_element_type=jnp.float32) + bf2_ref[...])
    o_ref[...] = jnp.dot(h, wo_ref[...],
                         preferred_element_type=jnp.float32) + bo_ref[...]


def _readout(s, weights):
    return pl.pallas_call(
        _readout_kernel,
        grid=(N // NB,),
        in_specs=[pl.BlockSpec((NB, F), lambda i: (i, 0)),
                  pl.BlockSpec((F, F), lambda i: (0, 0)),
                  pl.BlockSpec((1, F), lambda i: (0, 0)),
                  pl.BlockSpec((F, OUT_FEA), lambda i: (0, 0)),
                  pl.BlockSpec((1, OUT_FEA), lambda i: (0, 0)),
                  pl.BlockSpec((OUT_FEA, F), lambda i: (0, 0)),
                  pl.BlockSpec((1, F), lambda i: (0, 0)),
                  pl.BlockSpec((F, F), lambda i: (0, 0)),
                  pl.BlockSpec((1, F), lambda i: (0, 0)),
                  pl.BlockSpec((F, 8), lambda i: (0, 0)),
                  pl.BlockSpec((1, 8), lambda i: (0, 0))],
        out_specs=pl.BlockSpec((NB, 8), lambda i: (i, 0)),
        out_shape=jax.ShapeDtypeStruct((N, 8), jnp.float32),
    )(s, *weights)


def _sc_gather(table, idx3):
    rows_w = idx3.shape[1]
    d = table.shape[1]
    mesh = plsc.VectorSubcoreMesh(core_axis_name="c", subcore_axis_name="s")

    @functools.partial(
        pl.kernel,
        out_type=jax.ShapeDtypeStruct((NW * rows_w * GB, d), jnp.float32),
        mesh=mesh,
        scratch_types=[pltpu.VMEM((rows_w, GB), jnp.int32),
                       pltpu.VMEM((GB, d), jnp.float32),
                       pltpu.VMEM((GB, d), jnp.float32),
                       pltpu.SemaphoreType.DMA,
                       pltpu.SemaphoreType.DMA],
    )
    def k(table_hbm, idx_hbm, out_hbm, idx_v, buf0, buf1, sem0, sem1):
        wid = lax.axis_index("s") * 2 + lax.axis_index("c")
        r0 = wid * rows_w
        pltpu.sync_copy(idx_hbm.at[wid], idx_v)
        bufs = (buf0, buf1)
        sems = (sem0, sem1)

        @pl.loop(0, rows_w // 2)
        def _(jj):
            for t in range(2):
                j = jj * 2 + t
                e0 = (r0 + j) * GB
                pltpu.async_copy(table_hbm.at[idx_v.at[j]],
                                 bufs[t], sems[t]).wait()
                pltpu.sync_copy(bufs[t], out_hbm.at[pl.ds(e0, GB)])

    return k(table, idx3)


NACC = 10240


IGRP = 4


def _sc_scatter_sum(msg, dst4, zrows):
    ngrp = dst4.shape[1]
    rows_w = ngrp * IGRP
    nrow_sub = NACC // NSUB
    mesh = plsc.VectorSubcoreMesh(core_axis_name="c", subcore_axis_name="s")

    @functools.partial(
        pl.kernel,
        out_type=jax.ShapeDtypeStruct((4, NACC, F), jnp.float32),
        mesh=mesh,
        scratch_types=[pltpu.VMEM((IGRP, GB), jnp.int32),
                       pltpu.VMEM((GB, F), jnp.float32),
                       pltpu.VMEM_SHARED((NACC, F), jnp.float32)],
    )
    def k(msg_hbm, dst_hbm, z_hbm, out_hbm, idx_v, dbuf, acc):
        sid = lax.axis_index("s")
        cid = lax.axis_index("c")
        r0 = sid * rows_w
        n0 = sid * nrow_sub
        for p in range(2):
            chunk = cid * 2 + p
            pltpu.sync_copy(z_hbm.at[pl.ds(n0, nrow_sub)],
                            acc.at[pl.ds(n0, nrow_sub)])
            plsc.subcore_barrier()

            @pl.loop(0, ngrp)
            def _(jj):
                pltpu.sync_copy(dst_hbm.at[sid, jj], idx_v)
                for t in range(IGRP):
                    e0 = (r0 + jj * IGRP + t) * GB
                    pltpu.sync_copy(
                        msg_hbm.at[pl.ds(e0, GB), pl.ds(chunk * F, F)], dbuf)
                    pltpu.sync_copy(dbuf, acc.at[idx_v.at[t]], add=True)

            plsc.subcore_barrier()
            pltpu.sync_copy(acc.at[pl.ds(n0, nrow_sub)],
                            out_hbm.at[chunk, pl.ds(n0, nrow_sub)])
            plsc.subcore_barrier()

    return k(msg, dst4, zrows)


def kernel(nxyz, nbr_list, params):
    f32 = jnp.float32
    dst = jnp.concatenate([nbr_list[:, 0], nbr_list[:, 1]])
    src = jnp.concatenate([nbr_list[:, 1], nbr_list[:, 0]])
    src3 = src.reshape(NW, E2 // (NW * GB), GB)
    dst4 = dst.reshape(NSUB, E2 // (NSUB * GB * IGRP), IGRP, GB)
    src3_u = src[:EU].reshape(NW, EU // (NW * GB), GB)
    dst3_u = dst[:EU].reshape(NW, EU // (NW * GB), GB)

    xyz128 = jnp.zeros((N, 128), f32).at[:, 0:3].set(nxyz[:, 1:4])
    emb_pad = jnp.zeros((F, F), f32).at[0:100, :].set(params["embed"])
    zrows = jnp.zeros((NACC, F), f32)

    xyz_s = _sc_gather(xyz128, src3_u)
    xyz_d = _sc_gather(xyz128, dst3_u)
    p_u, unit_u = _geometry(xyz_s, xyz_d)

    s = _embed(nxyz, emb_pad)
    v = None

    for li, lp in enumerate(params["conv"]):
        wd = jnp.zeros((32, 3 * F), f32)
        wd = wd.at[0:RBF, :].set(lp["dist"]["W"])
        wd = wd.at[RBF, :].set(lp["dist"]["b"])
        phi = _phi(s, lp["phi1"]["W"], lp["phi1"]["b"].reshape(1, F),
                   lp["phi2"]["W"], lp["phi2"]["b"].reshape(1, 3 * F))
        phig = _sc_gather(phi, src3)
        vg = _sc_gather(v, src3) if li > 0 else None
        msg = _edge_messages(p_u, unit_u, wd, phig, vg)
        msum = _sc_scatter_sum(msg, dst4, zrows)[:, :N]
        if v is None:
            v = jnp.zeros((N, 3 * F), f32)
        s, v = _update(s, v, msum,
                       lp["U"]["W"], lp["V"]["W"],
                       lp["upd1"]["W"], lp["upd1"]["b"].reshape(1, F),
                       lp["upd2"]["W"], lp["upd2"]["b"].reshape(1, 3 * F))

    wo_pad = jnp.zeros((F, 8), f32).at[:, 0:1].set(params["fc_out"]["W"])
    bo_pad = jnp.zeros((1, 8), f32).at[0, 0].set(params["fc_out"]["b"][0])
    ro = _readout(s, [
        params["read1"]["W"], params["read1"]["b"].reshape(1, F),
        params["read2"]["W"], params["read2"]["b"].reshape(1, OUT_FEA),
        params["fc"][0]["W"], params["fc"][0]["b"].reshape(1, F),
        params["fc"][1]["W"], params["fc"][1]["b"].reshape(1, F),
        wo_pad, bo_pad,
    ])
    return ro[:, 0:1]


# speedup vs baseline: 21.2385x; 1.2693x over previous
import functools

import jax
import jax.numpy as jnp
from jax import lax
from jax.experimental import pallas as pl
from jax.experimental.pallas import tpu as pltpu
from jax.experimental.pallas import tpu_sc as plsc

N = 10000
EU = 320000
E2 = 2 * EU
F = 128
RBF = 20
CUT = 5.0
OUT_FEA = 64

NB = 400
EB = 512
GB = 80
NW = 32
NSUB = 16


def _silu(x):
    return x / (1.0 + jnp.exp(-x))


def _embed_kernel(nxyz_ref, emb_ref, o_ref):
    z = nxyz_ref[:, 0].astype(jnp.int32)
    cols = lax.broadcasted_iota(jnp.int32, (NB, F), 1)
    onehot = (cols == z[:, None]).astype(jnp.float32)
    o_ref[...] = jnp.dot(onehot, emb_ref[...],
                         preferred_element_type=jnp.float32)


def _embed(nxyz, emb_pad):
    return pl.pallas_call(
        _embed_kernel,
        grid=(N // NB,),
        in_specs=[pl.BlockSpec((NB, 4), lambda i: (i, 0)),
                  pl.BlockSpec((F, F), lambda i: (0, 0))],
        out_specs=pl.BlockSpec((NB, F), lambda i: (i, 0)),
        out_shape=jax.ShapeDtypeStruct((N, F), jnp.float32),
    )(nxyz, emb_pad)


def _geom_kernel(xs_ref, xd_ref, p_ref, u_ref):
    r8 = xs_ref[...] - xd_ref[...]
    d2 = jnp.sum(r8 * r8, axis=1, keepdims=True)
    dist = jnp.sqrt(d2 + 1e-15)
    u_ref[...] = (r8 / dist)[:, 0:16]
    fc = 0.5 * (jnp.cos(jnp.pi * dist / CUT) + 1.0)
    fc = fc * (dist <= CUT).astype(jnp.float32)
    k = lax.broadcasted_iota(jnp.int32, (EB, 32), 1)
    nfreq = (k + 1).astype(jnp.float32)
    rbf = jnp.sin(nfreq * (jnp.pi / CUT) * dist) / dist * fc
    p_ref[...] = jnp.where(k < RBF, rbf, jnp.where(k == RBF, fc, 0.0))


def _geometry(xyz_sd):
    nblk = EU // EB
    return pl.pallas_call(
        _geom_kernel,
        grid=(nblk,),
        in_specs=[pl.BlockSpec((EB, 128), lambda i: (i, 0)),
                  pl.BlockSpec((EB, 128), lambda i: (i + nblk, 0))],
        out_specs=[pl.BlockSpec((EB, 32), lambda i: (i, 0)),
                   pl.BlockSpec((EB, 16), lambda i: (i, 0))],
        out_shape=[jax.ShapeDtypeStruct((EU, 32), jnp.float32),
                   jax.ShapeDtypeStruct((EU, 16), jnp.float32)],
    )(xyz_sd, xyz_sd)


def _phi_kernel(s_ref, w1_ref, b1_ref, w2_ref, b2_ref, o_ref):
    h = _silu(jnp.dot(s_ref[...], w1_ref[...],
                      preferred_element_type=jnp.float32) + b1_ref[...])
    o_ref[...] = jnp.dot(h, w2_ref[...],
                         preferred_element_type=jnp.float32) + b2_ref[...]


def _phi(s, w1, b1, w2, b2):
    return pl.pallas_call(
        _phi_kernel,
        grid=(N // NB,),
        in_specs=[pl.BlockSpec((NB, F), lambda i: (i, 0)),
                  pl.BlockSpec((F, F), lambda i: (0, 0)),
                  pl.BlockSpec((1, F), lambda i: (0, 0)),
                  pl.BlockSpec((F, 3 * F), lambda i: (0, 0)),
                  pl.BlockSpec((1, 3 * F), lambda i: (0, 0))],
        out_specs=pl.BlockSpec((NB, 3 * F), lambda i: (i, 0)),
        out_shape=jax.ShapeDtypeStruct((N, 3 * F), jnp.float32),
    )(s, w1, b1, w2, b2)


def _edge_kernel(has_v, p_ref, u_ref, wd_ref, phig_ref, *rest):
    if has_v:
        vg_ref, m_ref = rest
    else:
        (m_ref,) = rest
    nblk = EU // EB
    sign = jnp.where(pl.program_id(0) < nblk, 1.0, -1.0)
    w_s = jnp.dot(p_ref[...], wd_ref[...],
                  preferred_element_type=jnp.float32)
    phig = phig_ref[...]
    inv0 = phig[:, 0:F] * w_s[:, 0:F]
    inv1 = phig[:, F:2 * F] * w_s[:, F:2 * F]
    inv2 = phig[:, 2 * F:3 * F] * w_s[:, 2 * F:3 * F]
    m_ref[:, 0:F] = inv1
    for d in range(3):
        unit_d = sign * u_ref[:, d][:, None]
        dv = inv2 * unit_d
        if has_v:
            dv = dv + inv0 * vg_ref[:, d * F:(d + 1) * F]
        m_ref[:, (d + 1) * F:(d + 2) * F] = dv


def _edge_messages(p_u, unit_u, wd, phig, vg):
    nblk = EU // EB
    has_v = vg is not None
    in_specs = [pl.BlockSpec((EB, 32), lambda i: (i % nblk, 0)),
                pl.BlockSpec((EB, 16), lambda i: (i % nblk, 0)),
                pl.BlockSpec((32, 3 * F), lambda i: (0, 0)),
                pl.BlockSpec((EB, 3 * F), lambda i: (i, 0))]
    args = [p_u, unit_u, wd, phig]
    if has_v:
        in_specs.append(pl.BlockSpec((EB, 3 * F), lambda i: (i, 0)))
        args.append(vg)
    return pl.pallas_call(
        functools.partial(_edge_kernel, has_v),
        grid=(E2 // EB,),
        in_specs=in_specs,
        out_specs=pl.BlockSpec((EB, 4 * F), lambda i: (i, 0)),
        out_shape=jax.ShapeDtypeStruct((E2, 4 * F), jnp.float32),
    )(*args)


def _update_kernel(s_ref, v_ref, m_ref, u_w_ref, v_w_ref,
                   w1_ref, b1_ref, w2_ref, b2_ref, so_ref, vo_ref):
    m = m_ref[...]
    s1 = s_ref[...] + m[0]
    v1 = [v_ref[:, d * F:(d + 1) * F] + m[1 + d] for d in range(3)]
    u_w = u_w_ref[...]
    v_w = v_w_ref[...]
    u_v = [jnp.dot(v1[d], u_w, preferred_element_type=jnp.float32)
           for d in range(3)]
    v_v = [jnp.dot(v1[d], v_w, preferred_element_type=jnp.float32)
           for d in range(3)]
    vvn = jnp.sqrt(v_v[0] * v_v[0] + v_v[1] * v_v[1] + v_v[2] * v_v[2]
                   + 1e-15)
    stack = jnp.concatenate([s1, vvn], axis=1)
    a = _silu(jnp.dot(stack, w1_ref[...],
                      preferred_element_type=jnp.float32) + b1_ref[...])
    split = jnp.dot(a, w2_ref[...],
                    preferred_element_type=jnp.float32) + b2_ref[...]
    s0 = split[:, 0:F]
    dot_uv = u_v[0] * v_v[0] + u_v[1] * v_v[1] + u_v[2] * v_v[2]
    so_ref[...] = s1 + split[:, F:2 * F] * dot_uv + split[:, 2 * F:3 * F]
    for d in range(3):
        vo_ref[:, d * F:(d + 1) * F] = v1[d] + u_v[d] * s0


def _update(s, v, msum, u_w, v_w, w1, b1, w2, b2):
    return pl.pallas_call(
        _update_kernel,
        grid=(N // NB,),
        in_specs=[pl.BlockSpec((NB, F), lambda i: (i, 0)),
                  pl.BlockSpec((NB, 3 * F), lambda i: (i, 0)),
                  pl.BlockSpec((4, NB, F), lambda i: (0, i, 0)),

                  pl.BlockSpec((F, F), lambda i: (0, 0)),
                  pl.BlockSpec((F, F), lambda i: (0, 0)),
                  pl.BlockSpec((2 * F, F), lambda i: (0, 0)),
                  pl.BlockSpec((1, F), lambda i: (0, 0)),
                  pl.BlockSpec((F, 3 * F), lambda i: (0, 0)),
                  pl.BlockSpec((1, 3 * F), lambda i: (0, 0))],
        out_specs=[pl.BlockSpec((NB, F), lambda i: (i, 0)),
                   pl.BlockSpec((NB, 3 * F), lambda i: (i, 0))],
        out_shape=[jax.ShapeDtypeStruct((N, F), jnp.float32),
                   jax.ShapeDtypeStruct((N, 3 * F), jnp.float32)],
    )(s, v, msum, u_w, v_w, w1, b1, w2, b2)


def _readout_kernel(s_ref, wr1_ref, br1_ref, wr2_ref, br2_ref,
                    wf1_ref, bf1_ref, wf2_ref, bf2_ref,
                    wo_ref, bo_ref, o_ref):
    h = _silu(jnp.dot(s_ref[...], wr1_ref[...],
                      preferred_element_type=jnp.float32) + br1_ref[...])
    af = jnp.dot(h, wr2_ref[...],
                 preferred_element_type=jnp.float32) + br2_ref[...]
    h = _silu(jnp.dot(af, wf1_ref[...],
                      preferred_element_type=jnp.float32) + bf1_ref[...])
    h = _silu(jnp.dot(h, wf2_ref[...],
                      preferred_element_type=jnp.float32) + bf2_ref[...])
    o_ref[...] = jnp.dot(h, wo_ref[...],
                         preferred_element_type=jnp.float32) + bo_ref[...]


def _readout(s, weights):
    return pl.pallas_call(
        _readout_kernel,
        grid=(N // NB,),
        in_specs=[pl.BlockSpec((NB, F), lambda i: (i, 0)),
                  pl.BlockSpec((F, F), lambda i: (0, 0)),
                  pl.BlockSpec((1, F), lambda i: (0, 0)),
                  pl.BlockSpec((F, OUT_FEA), lambda i: (0, 0)),
                  pl.BlockSpec((1, OUT_FEA), lambda i: (0, 0)),
                  pl.BlockSpec((OUT_FEA, F), lambda i: (0, 0)),
                  pl.BlockSpec((1, F), lambda i: (0, 0)),
                  pl.BlockSpec((F, F), lambda i: (0, 0)),
                  pl.BlockSpec((1, F), lambda i: (0, 0)),
                  pl.BlockSpec((F, 8), lambda i: (0, 0)),
                  pl.BlockSpec((1, 8), lambda i: (0, 0))],
        out_specs=pl.BlockSpec((NB, 8), lambda i: (i, 0)),
        out_shape=jax.ShapeDtypeStruct((N, 8), jnp.float32),
    )(s, *weights)


def _sc_gather(table, idx3):
    rows_w = idx3.shape[1]
    d = table.shape[1]
    mesh = plsc.VectorSubcoreMesh(core_axis_name="c", subcore_axis_name="s")

    @functools.partial(
        pl.kernel,
        out_type=jax.ShapeDtypeStruct((NW * rows_w * GB, d), jnp.float32),
        mesh=mesh,
        scratch_types=[pltpu.VMEM((rows_w, GB), jnp.int32),
                       pltpu.VMEM((GB, d), jnp.float32),
                       pltpu.VMEM((GB, d), jnp.float32),
                       pltpu.SemaphoreType.DMA,
                       pltpu.SemaphoreType.DMA,
                       pltpu.SemaphoreType.DMA,
                       pltpu.SemaphoreType.DMA],
    )
    def k(table_hbm, idx_hbm, out_hbm, idx_v, bufa, bufb, ga, gb, wa, wb):
        wid = lax.axis_index("s") * 2 + lax.axis_index("c")
        r0 = wid * rows_w
        pltpu.sync_copy(idx_hbm.at[wid], idx_v)

        def g_start(j, buf, sem):
            pltpu.make_async_copy(table_hbm.at[idx_v.at[j]], buf, sem).start()

        def g_wait(j, buf, sem):
            pltpu.make_async_copy(table_hbm.at[idx_v.at[j]], buf, sem).wait()

        def w_start(j, buf, sem):
            pltpu.make_async_copy(
                buf, out_hbm.at[pl.ds((r0 + j) * GB, GB)], sem).start()

        def w_wait(j, buf, sem):
            pltpu.make_async_copy(
                buf, out_hbm.at[pl.ds((r0 + j) * GB, GB)], sem).wait()

        g_start(0, bufa, ga)
        g_start(1, bufb, gb)
        g_wait(0, bufa, ga)
        w_start(0, bufa, wa)
        g_wait(1, bufb, gb)
        w_start(1, bufb, wb)

        @pl.loop(1, rows_w // 2)
        def _(i):
            j0 = 2 * i
            j1 = 2 * i + 1
            w_wait(j0 - 2, bufa, wa)
            g_start(j0, bufa, ga)
            w_wait(j1 - 2, bufb, wb)
            g_start(j1, bufb, gb)
            g_wait(j0, bufa, ga)
            w_start(j0, bufa, wa)
            g_wait(j1, bufb, gb)
            w_start(j1, bufb, wb)

        w_wait(rows_w - 2, bufa, wa)
        w_wait(rows_w - 1, bufb, wb)

    return k(table, idx3)


NACC = 10240


IGRP = 20


def _sc_scatter_sum(msg, dst4, zrows):
    ngrp = dst4.shape[1]
    rows_w = ngrp * IGRP
    nrow_sub = NACC // NSUB
    mesh = plsc.VectorSubcoreMesh(core_axis_name="c", subcore_axis_name="s")

    @functools.partial(
        pl.kernel,
        out_type=jax.ShapeDtypeStruct((4, NACC, F), jnp.float32),
        mesh=mesh,
        scratch_types=[pltpu.VMEM((IGRP, GB), jnp.int32),
                       pltpu.VMEM((GB, F), jnp.float32),
                       pltpu.VMEM((GB, F), jnp.float32),
                       pltpu.SemaphoreType.DMA,
                       pltpu.SemaphoreType.DMA,
                       pltpu.VMEM_SHARED((NACC, F), jnp.float32)],
    )
    def k(msg_hbm, dst_hbm, z_hbm, out_hbm, idx_v, bufa, bufb, la, lb, acc):
        sid = lax.axis_index("s")
        cid = lax.axis_index("c")
        r0 = sid * rows_w
        n0 = sid * nrow_sub
        last = rows_w - 1

        def l_start(chunk, j, buf, sem):
            pltpu.make_async_copy(
                msg_hbm.at[pl.ds((r0 + j) * GB, GB), pl.ds(chunk * F, F)],
                buf, sem).start()

        def l_wait(chunk, j, buf, sem):
            pltpu.make_async_copy(
                msg_hbm.at[pl.ds((r0 + j) * GB, GB), pl.ds(chunk * F, F)],
                buf, sem).wait()

        for p in range(2):
            chunk = cid * 2 + p
            pltpu.sync_copy(z_hbm.at[pl.ds(n0, nrow_sub)],
                            acc.at[pl.ds(n0, nrow_sub)])
            plsc.subcore_barrier()
            l_start(chunk, 0, bufa, la)

            @pl.loop(0, ngrp)
            def _(g):
                pltpu.sync_copy(dst_hbm.at[sid, g], idx_v)
                for t in range(IGRP):
                    j = g * IGRP + t
                    cur, csem = (bufa, la) if t % 2 == 0 else (bufb, lb)
                    nxt, nsem = (bufb, lb) if t % 2 == 0 else (bufa, la)
                    jn = jnp.minimum(j + 1, last)
                    l_start(chunk, jn, nxt, nsem)
                    l_wait(chunk, j, cur, csem)
                    pltpu.sync_copy(cur, acc.at[idx_v.at[t]], add=True)

            l_wait(chunk, last, bufa, la)
            plsc.subcore_barrier()
            pltpu.sync_copy(acc.at[pl.ds(n0, nrow_sub)],
                            out_hbm.at[chunk, pl.ds(n0, nrow_sub)])
            plsc.subcore_barrier()

    return k(msg, dst4, zrows)


def kernel(nxyz, nbr_list, params):
    f32 = jnp.float32
    dst = jnp.concatenate([nbr_list[:, 0], nbr_list[:, 1]])
    src = jnp.concatenate([nbr_list[:, 1], nbr_list[:, 0]])
    src3 = src.reshape(NW, E2 // (NW * GB), GB)
    dst4 = dst.reshape(NSUB, E2 // (NSUB * GB * IGRP), IGRP, GB)
    sd_u = jnp.concatenate([src[:EU], dst[:EU]])
    sd3_u = sd_u.reshape(NW, E2 // (NW * GB), GB)

    xyz128 = jnp.zeros((N, 128), f32).at[:, 0:3].set(nxyz[:, 1:4])
    emb_pad = jnp.zeros((F, F), f32).at[0:100, :].set(params["embed"])
    zrows = jnp.zeros((NACC, F), f32)

    xyz_sd = _sc_gather(xyz128, sd3_u)
    p_u, unit_u = _geometry(xyz_sd)

    s = _embed(nxyz, emb_pad)
    v = None

    for li, lp in enumerate(params["conv"]):
        wd = jnp.zeros((32, 3 * F), f32)
        wd = wd.at[0:RBF, :].set(lp["dist"]["W"])
        wd = wd.at[RBF, :].set(lp["dist"]["b"])
        phi = _phi(s, lp["phi1"]["W"], lp["phi1"]["b"].reshape(1, F),
                   lp["phi2"]["W"], lp["phi2"]["b"].reshape(1, 3 * F))
        phig = _sc_gather(phi, src3)
        vg = _sc_gather(v, src3) if li > 0 else None
        msg = _edge_messages(p_u, unit_u, wd, phig, vg)
        msum = _sc_scatter_sum(msg, dst4, zrows)
        if v is None:
            v = jnp.zeros((N, 3 * F), f32)
        s, v = _update(s, v, msum,
                       lp["U"]["W"], lp["V"]["W"],
                       lp["upd1"]["W"], lp["upd1"]["b"].reshape(1, F),
                       lp["upd2"]["W"], lp["upd2"]["b"].reshape(1, 3 * F))

    wo_pad = jnp.zeros((F, 8), f32).at[:, 0:1].set(params["fc_out"]["W"])
    bo_pad = jnp.zeros((1, 8), f32).at[0, 0].set(params["fc_out"]["b"][0])
    ro = _readout(s, [
        params["read1"]["W"], params["read1"]["b"].reshape(1, F),
        params["read2"]["W"], params["read2"]["b"].reshape(1, OUT_FEA),
        params["fc"][0]["W"], params["fc"][0]["b"].reshape(1, F),
        params["fc"][1]["W"], params["fc"][1]["b"].reshape(1, F),
        wo_pad, bo_pad,
    ])
    return ro[:, 0:1]
